# Initial kernel scaffold; baseline (speedup 1.0000x reference)
import jax
import jax.numpy as jnp
from jax.experimental import pallas as pl


def kernel(inputs, params):
    raise NotImplementedError("write your pallas kernel here")


# speedup vs baseline: 2.9085x; 2.9085x over previous
import functools
import jax
import jax.numpy as jnp
import numpy as np
from jax.experimental import pallas as pl

C = 768
K = 9
N = 3136
B = 2
ROWS = 392


def _conv2d(x, W, b=None, stride=1, pad=0):
    out = jax.lax.conv_general_dilated(
        x, W, (stride, stride), ((pad, pad), (pad, pad)),
        dimension_numbers=('NCHW', 'OIHW', 'NCHW'))
    if b is not None:
        out = out + b[None, :, None, None]
    return out


def _bn_nchw(x, g, b, eps=1e-5):
    m = jnp.mean(x, axis=(0, 2, 3), keepdims=True)
    v = jnp.var(x, axis=(0, 2, 3), keepdims=True)
    return (x - m) / jnp.sqrt(v + eps) * g[None, :, None, None] + b[None, :, None, None]


def _knn_body(f_rows_ref, f_full_ref, sq_full_ref, idx_ref):
    frf = f_rows_ref[0]
    fr = frf.astype(jnp.bfloat16)
    ff = f_full_ref[0].astype(jnp.bfloat16)
    sqr = jnp.sum(frf * frf, axis=1, keepdims=True)
    sqf = sq_full_ref[0, 0]
    dot = jax.lax.dot_general(fr, ff, (((1,), (1,)), ((), ())),
                              preferred_element_type=jnp.float32)
    s = sqr + sqf[None, :] - 2.0 * dot
    cols = jax.lax.broadcasted_iota(jnp.int32, s.shape, 1)
    picks = []
    for _ in range(K):
        m = jnp.min(s, axis=1, keepdims=True)
        cand = jnp.where(s == m, cols, jnp.int32(2 ** 30))
        ik = jnp.min(cand, axis=1, keepdims=True)
        picks.append(ik)
        s = jnp.where(cols == ik, jnp.float32(3e38), s)
    picks.append(jnp.zeros((ROWS, 16 - K), jnp.int32))
    idx_ref[0] = jnp.concatenate(picks, axis=1)


def _knn_topk(f):
    sq = jnp.sum(f * f, axis=-1)
    idx16 = pl.pallas_call(
        _knn_body,
        grid=(B, N // ROWS),
        in_specs=[
            pl.BlockSpec((1, ROWS, C), lambda b, i: (b, i, 0)),
            pl.BlockSpec((1, N, C), lambda b, i: (b, 0, 0)),
            pl.BlockSpec((1, 1, N), lambda b, i: (b, 0, 0)),
        ],
        out_specs=pl.BlockSpec((1, ROWS, 16), lambda b, i: (b, i, 0)),
        out_shape=jax.ShapeDtypeStruct((B, N, 16), jnp.int32),
    )(f, f, sq[:, None, :])
    return idx16[:, :, :K]


def _gather_reduce(Q, P, idx):
    bidx = jnp.arange(B)[:, None, None]
    Qg = Q[bidx, idx]
    G1 = jnp.sum(Qg, axis=2)
    Mx = jnp.max(Qg, axis=2)
    Mn = jnp.min(Qg, axis=2)
    deg = jnp.zeros((B, N), jnp.float32).at[bidx, idx].add(1.0)
    T = jnp.sum(deg[..., None] * Q * Q, axis=(0, 1))
    dotPG = jnp.sum(P * G1, axis=(0, 1))
    sumG1 = jnp.sum(G1, axis=(0, 1))
    return Mx, Mn, sumG1, dotPG, T


def kernel(inputs, params):
    p = params
    x = _conv2d(inputs, p['s1w'], p['s1b'], stride=2, pad=1)
    x = jax.nn.relu(_bn_nchw(x, p['s1g'], p['s1be']))
    x = _conv2d(x, p['s2w'], p['s2b'], stride=2, pad=1)
    x = jax.nn.relu(_bn_nchw(x, p['s2g'], p['s2be']))
    x = _conv2d(x, p['s3w'], p['s3b'], stride=1, pad=1)
    x = _bn_nchw(x, p['s3g'], p['s3be'])
    x = x + p['pos']

    shortcut = x
    y = _conv2d(x, p['gf1w'], p['gf1b'])
    y = _bn_nchw(y, p['gf1g'], p['gf1be'])
    Bc, Cc, H, W = y.shape
    f = y.reshape(Bc, Cc, N).transpose(0, 2, 1)

    idx = _knn_topk(f)

    xj = jax.vmap(lambda fb, ib: fb[ib])(f, idx)
    xi = jnp.broadcast_to(f[:, :, None, :], xj.shape)
    feat = jnp.concatenate([xi, xj - xi], axis=-1)
    e = jnp.einsum('oi,bnki->bnko', p['gecw'], feat) + p['gecb']
    m = jnp.mean(e, axis=(0, 1, 2))
    v = jnp.var(e, axis=(0, 1, 2))
    z = (e - m) / jnp.sqrt(v + 1e-5) * p['gecg'] + p['gecbe']
    e = jnp.max(jax.nn.gelu(z), axis=2)

    e = e.transpose(0, 2, 1).reshape(Bc, 2 * Cc, H, W)
    y = _conv2d(e, p['gf2w'], p['gf2b'])
    y = _bn_nchw(y, p['gf2g'], p['gf2be'])
    x = y + shortcut

    shortcut = x
    y = _conv2d(x, p['ff1w'], p['ff1b'])
    y = jax.nn.gelu(_bn_nchw(y, p['ff1g'], p['ff1be']))
    y = _conv2d(y, p['ff2w'], p['ff2b'])
    y = _bn_nchw(y, p['ff2g'], p['ff2be'])
    x = y + shortcut

    x = jnp.max(x, axis=(2, 3), keepdims=True)
    x = _conv2d(x, p['c1w'], p['c1b'])
    x = jax.nn.relu(_bn_nchw(x, p['c1g'], p['c1be']))
    x = _conv2d(x, p['c2w'], p['c2b'])
    return x[:, :, 0, 0]


# speedup vs baseline: 4.1294x; 1.4198x over previous
import functools
import jax
import jax.numpy as jnp
import numpy as np
from jax import lax
from jax.experimental import pallas as pl
from jax.experimental.pallas import tpu as pltpu
from jax.experimental.pallas import tpu_sc as plsc

C = 768
K = 9
N = 3136
B = 2
ROWS = 392


def _conv2d(x, W, b=None, stride=1, pad=0):
    out = jax.lax.conv_general_dilated(
        x, W, (stride, stride), ((pad, pad), (pad, pad)),
        dimension_numbers=('NCHW', 'OIHW', 'NCHW'))
    if b is not None:
        out = out + b[None, :, None, None]
    return out


def _bn_nchw(x, g, b, eps=1e-5):
    m = jnp.mean(x, axis=(0, 2, 3), keepdims=True)
    v = jnp.var(x, axis=(0, 2, 3), keepdims=True)
    return (x - m) / jnp.sqrt(v + eps) * g[None, :, None, None] + b[None, :, None, None]


def _knn_body(f_rows_ref, f_full_ref, sq_full_ref, idx_ref):
    frf = f_rows_ref[0]
    fr = frf.astype(jnp.bfloat16)
    ff = f_full_ref[0].astype(jnp.bfloat16)
    sqr = jnp.sum(frf * frf, axis=1, keepdims=True)
    sqf = sq_full_ref[0, 0]
    dot = jax.lax.dot_general(fr, ff, (((1,), (1,)), ((), ())),
                              preferred_element_type=jnp.float32)
    s = sqr + sqf[None, :] - 2.0 * dot
    cols = jax.lax.broadcasted_iota(jnp.int32, s.shape, 1)
    picks = []
    for _ in range(K):
        m = jnp.min(s, axis=1, keepdims=True)
        cand = jnp.where(s == m, cols, jnp.int32(2 ** 30))
        ik = jnp.min(cand, axis=1, keepdims=True)
        picks.append(ik)
        s = jnp.where(cols == ik, jnp.float32(3e38), s)
    picks.append(jnp.zeros((ROWS, 16 - K), jnp.int32))
    idx_ref[0] = jnp.concatenate(picks, axis=1)


def _knn_topk(f):
    sq = jnp.sum(f * f, axis=-1)
    idx16 = pl.pallas_call(
        _knn_body,
        grid=(B, N // ROWS),
        in_specs=[
            pl.BlockSpec((1, ROWS, C), lambda b, i: (b, i, 0)),
            pl.BlockSpec((1, N, C), lambda b, i: (b, 0, 0)),
            pl.BlockSpec((1, 1, N), lambda b, i: (b, 0, 0)),
        ],
        out_specs=pl.BlockSpec((1, ROWS, 16), lambda b, i: (b, i, 0)),
        out_shape=jax.ShapeDtypeStruct((B, N, 16), jnp.int32),
    )(f, f, sq[:, None, :])
    return idx16[:, :, :K]


TWO_C = 2 * C
_NW = 32
_NPW = (B * N) // _NW
_NCHUNK = TWO_C // 16
_G = 4
_GIDX = 40
_NGRP = _NPW // _G


def _sc_gather(Qf, idxp):
    mesh = plsc.VectorSubcoreMesh(core_axis_name="c", subcore_axis_name="s")
    info = plsc.get_sparse_core_info()
    nc = info.num_cores
    ngrp_total = (B * N) // _G

    @functools.partial(
        pl.kernel, mesh=mesh,
        compiler_params=pltpu.CompilerParams(use_tc_tiling_on_sc=False),
        out_type=[
            jax.ShapeDtypeStruct((ngrp_total, _G, TWO_C), jnp.float32),
            jax.ShapeDtypeStruct((ngrp_total, _G, TWO_C), jnp.float32),
            jax.ShapeDtypeStruct((ngrp_total, _G, TWO_C), jnp.float32),
            jax.ShapeDtypeStruct((_NW, 1, TWO_C), jnp.float32),
        ],
        scratch_types=[
            pltpu.VMEM((_NGRP * _GIDX,), jnp.int32),
            pltpu.VMEM((_GIDX, TWO_C), jnp.float32),
            pltpu.VMEM((_G, TWO_C), jnp.float32),
            pltpu.VMEM((_G, TWO_C), jnp.float32),
            pltpu.VMEM((_G, TWO_C), jnp.float32),
            pltpu.VMEM((1, TWO_C), jnp.float32),
            pltpu.SemaphoreType.DMA,
        ],
    )
    def k(q_hbm, idx_hbm, mx_hbm, mn_hbm, g1_hbm, a2_hbm,
          idx_v, rows_v, mx_v, mn_v, g1_v, a2_v, sem):
        wid = lax.axis_index("s") * nc + lax.axis_index("c")
        pltpu.sync_copy(idx_hbm.at[wid], idx_v)

        def zero_body(cc, _):
            a2_v[0, pl.ds(cc * 16, 16)] = jnp.zeros((16,), jnp.float32)
            return 0
        lax.fori_loop(0, _NCHUNK, zero_body, 0)

        def group_body(g, _):
            pltpu.async_copy(
                q_hbm.at[idx_v.at[pl.ds(g * _GIDX, _GIDX)]], rows_v, sem).wait()

            def chunk_body(cc, _):
                off = cc * 16
                a2c = a2_v[0, pl.ds(off, 16)]
                for j in range(_G):
                    v = rows_v[j * K, pl.ds(off, 16)]
                    mx = v
                    mn = v
                    s = v
                    a2c = a2c + v * v
                    for kk in range(1, K):
                        v = rows_v[j * K + kk, pl.ds(off, 16)]
                        mx = jnp.maximum(mx, v)
                        mn = jnp.minimum(mn, v)
                        s = s + v
                        a2c = a2c + v * v
                    mx_v[j, pl.ds(off, 16)] = mx
                    mn_v[j, pl.ds(off, 16)] = mn
                    g1_v[j, pl.ds(off, 16)] = s
                a2_v[0, pl.ds(off, 16)] = a2c
                return 0
            lax.fori_loop(0, _NCHUNK, chunk_body, 0)
            gg = wid * _NGRP + g
            pltpu.sync_copy(mx_v, mx_hbm.at[gg])
            pltpu.sync_copy(mn_v, mn_hbm.at[gg])
            pltpu.sync_copy(g1_v, g1_hbm.at[gg])
            return 0
        lax.fori_loop(0, _NGRP, group_body, 0)
        pltpu.sync_copy(a2_v, a2_hbm.at[wid])

    return k(Qf, idxp)


def _gather_reduce(Q, P, idx):
    Qf = Q.reshape(B * N, TWO_C)
    idxp = idx + (jnp.arange(B, dtype=jnp.int32) * N)[:, None, None]
    idxp = idxp.reshape(B * N // _G, _G * K)
    idxp = jnp.pad(idxp, ((0, 0), (0, _GIDX - _G * K)))
    mx, mn, g1, a2 = _sc_gather(Qf, idxp.astype(jnp.int32).reshape(_NW, _NGRP * _GIDX))
    Mx = mx.reshape(B, N, TWO_C)
    Mn = mn.reshape(B, N, TWO_C)
    g1 = g1.reshape(B * N, TWO_C)
    T = jnp.sum(a2[:, 0], axis=0)
    dotPG = jnp.sum(P.reshape(B * N, TWO_C) * g1, axis=0)
    sumG1 = jnp.sum(g1, axis=0)
    return Mx, Mn, sumG1, dotPG, T


def kernel(inputs, params):
    p = params
    x = _conv2d(inputs, p['s1w'], p['s1b'], stride=2, pad=1)
    x = jax.nn.relu(_bn_nchw(x, p['s1g'], p['s1be']))
    x = _conv2d(x, p['s2w'], p['s2b'], stride=2, pad=1)
    x = jax.nn.relu(_bn_nchw(x, p['s2g'], p['s2be']))
    x = _conv2d(x, p['s3w'], p['s3b'], stride=1, pad=1)
    x = _bn_nchw(x, p['s3g'], p['s3be'])
    x = x + p['pos']

    shortcut = x
    y = _conv2d(x, p['gf1w'], p['gf1b'])
    y = _bn_nchw(y, p['gf1g'], p['gf1be'])
    Bc, Cc, H, W = y.shape
    f = y.reshape(Bc, Cc, N).transpose(0, 2, 1)

    idx = _knn_topk(f)

    W1 = p['gecw'][:, :C]
    W2 = p['gecw'][:, C:]
    P = f @ (W1 - W2).T + p['gecb']
    Q = f @ W2.T

    Mx, Mn, sumG1, dotPG, T = _gather_reduce(Q, P, idx)

    cnt = np.float32(B * N * K)
    sumP = jnp.sum(P, axis=(0, 1))
    sumP2 = jnp.sum(P * P, axis=(0, 1))
    mu = (K * sumP + sumG1) / cnt
    var = (K * sumP2 + 2.0 * dotPG + T) / cnt - mu * mu
    sc = p['gecg'] / jnp.sqrt(var + 1e-5)
    sh = p['gecbe'] - mu * sc
    zA = (P + Mx) * sc + sh
    zB = (P + Mn) * sc + sh
    e = jnp.maximum(jax.nn.gelu(zA), jax.nn.gelu(zB))

    e = e.transpose(0, 2, 1).reshape(Bc, 2 * Cc, H, W)
    y = _conv2d(e, p['gf2w'], p['gf2b'])
    y = _bn_nchw(y, p['gf2g'], p['gf2be'])
    x = y + shortcut

    shortcut = x
    y = _conv2d(x, p['ff1w'], p['ff1b'])
    y = jax.nn.gelu(_bn_nchw(y, p['ff1g'], p['ff1be']))
    y = _conv2d(y, p['ff2w'], p['ff2b'])
    y = _bn_nchw(y, p['ff2g'], p['ff2be'])
    x = y + shortcut

    x = jnp.max(x, axis=(2, 3), keepdims=True)
    x = _conv2d(x, p['c1w'], p['c1b'])
    x = jax.nn.relu(_bn_nchw(x, p['c1g'], p['c1be']))
    x = _conv2d(x, p['c2w'], p['c2b'])
    return x[:, :, 0, 0]
